# baseline (speedup 1.0000x reference)
import functools

import jax
import jax.numpy as jnp
import numpy as np
from jax.experimental import pallas as pl
from jax.experimental.pallas import tpu as pltpu

_SIGMA = 0.2
_B = 64
_H = 28
_W = 28
_C = 384
_F = _H * _W * _C
_LANES = 128
_ROWS = _F // _LANES


def _consts():
    cpu = jax.devices("cpu")[0]
    with jax.default_device(cpu):
        key = jax.random.key(42)
        k1, k2, k3 = jax.random.split(key, 3)
        indices = jax.random.permutation(k1, _B)
        rs = (1, _H, _W, _C)
        r = jax.random.normal(k2, rs, dtype=jnp.float16) * jnp.float16(_SIGMA)
        theta = jax.random.uniform(
            k3, rs, dtype=jnp.float16, minval=-np.pi, maxval=np.pi)
        a = (jnp.float16(1.0) + r * jnp.cos(theta)).astype(jnp.float32)
        b = (r * jnp.sin(theta)).astype(jnp.float32)
        a_np = np.asarray(a).reshape(_ROWS, _LANES)
        b_np = np.asarray(b).reshape(_ROWS, _LANES)
        perm_np = np.asarray(indices, dtype=np.int32)
    return a_np, b_np, perm_np


_A_NP, _B_NP, _PERM_NP = _consts()


def _mix_body(perm_ref, xs_ref, xp_ref, a_ref, b_ref, out_ref):
    del perm_ref
    out_ref[0] = xs_ref[0] * a_ref[...] + xp_ref[0] * b_ref[...]


def kernel(x):
    x2 = x.reshape(_B, _ROWS, _LANES)
    a = jnp.asarray(_A_NP)
    b = jnp.asarray(_B_NP)
    perm = jnp.asarray(_PERM_NP)

    nchunk = 6
    crows = _ROWS // nchunk
    grid_spec = pltpu.PrefetchScalarGridSpec(
        num_scalar_prefetch=1,
        grid=(_B, nchunk),
        in_specs=[
            pl.BlockSpec((1, crows, _LANES), lambda i, j, p: (i, j, 0)),
            pl.BlockSpec((1, crows, _LANES), lambda i, j, p: (p[i], j, 0)),
            pl.BlockSpec((crows, _LANES), lambda i, j, p: (j, 0)),
            pl.BlockSpec((crows, _LANES), lambda i, j, p: (j, 0)),
        ],
        out_specs=pl.BlockSpec((1, crows, _LANES), lambda i, j, p: (i, j, 0)),
    )
    y2 = pl.pallas_call(
        _mix_body,
        grid_spec=grid_spec,
        out_shape=jax.ShapeDtypeStruct((_B, _ROWS, _LANES), jnp.float32),
        compiler_params=pltpu.CompilerParams(
            dimension_semantics=("arbitrary", "arbitrary")),
    )(perm, x2, x2, a, b)
    return y2.reshape(_B, _H, _W, _C)


# speedup vs baseline: 1.3244x; 1.3244x over previous
import functools

import jax
import jax.numpy as jnp
import numpy as np
from jax import lax
from jax.experimental import pallas as pl
from jax.experimental.pallas import tpu as pltpu
from jax.experimental.pallas import tpu_sc as plsc

_SIGMA = 0.2
_B = 64
_H = 28
_W = 28
_C = 384
_F = _H * _W * _C
_NW = 32
_FPW = _F // _NW
_LANES = 16
_NVEC = _FPW // _LANES


def _consts():
    cpu = jax.devices("cpu")[0]
    with jax.default_device(cpu):
        key = jax.random.key(42)
        k1, k2, k3 = jax.random.split(key, 3)
        indices = jax.random.permutation(k1, _B)
        rs = (1, _H, _W, _C)
        r = jax.random.normal(k2, rs, dtype=jnp.float16) * jnp.float16(_SIGMA)
        theta = jax.random.uniform(
            k3, rs, dtype=jnp.float16, minval=-np.pi, maxval=np.pi)
        a = (jnp.float16(1.0) + r * jnp.cos(theta)).astype(jnp.float32)
        b = (r * jnp.sin(theta)).astype(jnp.float32)
        a_np = np.asarray(a).reshape(_NW, _FPW)
        b_np = np.asarray(b).reshape(_NW, _FPW)
        perm_np = np.asarray(indices, dtype=np.int32)
    return a_np, b_np, perm_np


_A_NP, _B_NP, _PERM_NP = _consts()


def _mix_chunk(dst, xs, xp, av, bv):
    def body(i, c):
        base = i * (4 * _LANES)
        for u in range(4):
            sl = pl.ds(base + u * _LANES, _LANES)
            dst[sl] = xs[sl] * av[sl] + xp[sl] * bv[sl]
        return c
    lax.fori_loop(0, _NVEC // 4, body, 0, unroll=False)


def _sc_mix(x3, a2, b2):
    perm = [int(v) for v in _PERM_NP]
    mesh = plsc.VectorSubcoreMesh(core_axis_name="c", subcore_axis_name="s")

    @functools.partial(
        pl.kernel,
        out_type=jax.ShapeDtypeStruct((_B, _NW, _FPW), jnp.float32),
        mesh=mesh,
        scratch_types=[
            pltpu.VMEM((_FPW,), jnp.float32),
            pltpu.VMEM((_FPW,), jnp.float32),
            pltpu.VMEM((_FPW,), jnp.float32),
            pltpu.VMEM((_FPW,), jnp.float32),
            pltpu.VMEM((_FPW,), jnp.float32),
            pltpu.VMEM((_FPW,), jnp.float32),
            pltpu.VMEM((_FPW,), jnp.float32),
            pltpu.VMEM((_FPW,), jnp.float32),
            pltpu.SemaphoreType.DMA,
            pltpu.SemaphoreType.DMA,
            pltpu.SemaphoreType.DMA,
            pltpu.SemaphoreType.DMA,
            pltpu.SemaphoreType.DMA,
            pltpu.SemaphoreType.DMA,
        ],
    )
    def k(x_hbm, a_hbm, b_hbm, out_hbm,
          a_v, b_v, xs0, xp0, xs1, xp1, st0, st1,
          sxs0, sxp0, sxs1, sxp1, so0, so1):
        cid = lax.axis_index("c")
        sid = lax.axis_index("s")
        wid = sid * 2 + cid

        xs = (xs0, xs1)
        xp = (xp0, xp1)
        sxs = (sxs0, sxs1)
        sxp = (sxp0, sxp1)
        st = (st0, st1)
        so = (so0, so1)

        pltpu.sync_copy(a_hbm.at[wid], a_v)
        pltpu.sync_copy(b_hbm.at[wid], b_v)

        def start_fetch(t):
            j = t % 2
            pltpu.make_async_copy(x_hbm.at[t, wid], xs[j], sxs[j]).start()
            pltpu.make_async_copy(x_hbm.at[perm[t], wid], xp[j], sxp[j]).start()

        def wait_fetch(t):
            j = t % 2
            pltpu.make_async_copy(x_hbm.at[t, wid], xs[j], sxs[j]).wait()
            pltpu.make_async_copy(x_hbm.at[perm[t], wid], xp[j], sxp[j]).wait()

        start_fetch(0)
        for t in range(_B):
            j = t % 2
            if t + 1 < _B:
                start_fetch(t + 1)
            wait_fetch(t)
            if t >= 2:
                pltpu.make_async_copy(st[j], out_hbm.at[t - 2, wid], so[j]).wait()
            _mix_chunk(st[j], xs[j], xp[j], a_v, b_v)
            pltpu.make_async_copy(st[j], out_hbm.at[t, wid], so[j]).start()
        pltpu.make_async_copy(st[0], out_hbm.at[_B - 2, wid], so[0]).wait()
        pltpu.make_async_copy(st[1], out_hbm.at[_B - 1, wid], so[1]).wait()

    return k(x3, a2, b2)


def kernel(x):
    x3 = x.reshape(_B, _NW, _FPW)
    a2 = jnp.asarray(_A_NP)
    b2 = jnp.asarray(_B_NP)
    y3 = _sc_mix(x3, a2, b2)
    return y3.reshape(_B, _H, _W, _C)


# speedup vs baseline: 2.4604x; 1.8577x over previous
import jax
import jax.numpy as jnp
import numpy as np
from jax.experimental import pallas as pl
from jax.experimental.pallas import tpu as pltpu

_SIGMA = 0.2
_B = 64
_H = 28
_W = 28
_C = 384


def _consts():
    cpu = jax.devices("cpu")[0]
    with jax.default_device(cpu):
        key = jax.random.key(42)
        k1, k2, k3 = jax.random.split(key, 3)
        indices = jax.random.permutation(k1, _B)
        rs = (1, _H, _W, _C)
        r = jax.random.normal(k2, rs, dtype=jnp.float16) * jnp.float16(_SIGMA)
        theta = jax.random.uniform(
            k3, rs, dtype=jnp.float16, minval=-np.pi, maxval=np.pi)
        a = (jnp.float16(1.0) + r * jnp.cos(theta)).astype(jnp.float32)
        b = (r * jnp.sin(theta)).astype(jnp.float32)
        a_np = np.asarray(a).reshape(_H, _W, _C)
        b_np = np.asarray(b).reshape(_H, _W, _C)
        perm_np = np.asarray(indices, dtype=np.int32)
    return a_np, b_np, perm_np


_A_NP, _B_NP, _PERM_NP = _consts()


def _mix_body(perm_ref, xs_ref, xp_ref, a_ref, b_ref, out_ref):
    del perm_ref
    out_ref[0] = xs_ref[0] * a_ref[...] + xp_ref[0] * b_ref[...]


def kernel(x):
    a = jnp.asarray(_A_NP)
    b = jnp.asarray(_B_NP)
    perm = jnp.asarray(_PERM_NP)

    grid_spec = pltpu.PrefetchScalarGridSpec(
        num_scalar_prefetch=1,
        grid=(_B,),
        in_specs=[
            pl.BlockSpec((1, _H, _W, _C), lambda i, p: (i, 0, 0, 0)),
            pl.BlockSpec((1, _H, _W, _C), lambda i, p: (p[i], 0, 0, 0)),
            pl.BlockSpec((_H, _W, _C), lambda i, p: (0, 0, 0)),
            pl.BlockSpec((_H, _W, _C), lambda i, p: (0, 0, 0)),
        ],
        out_specs=pl.BlockSpec((1, _H, _W, _C), lambda i, p: (i, 0, 0, 0)),
    )
    y = pl.pallas_call(
        _mix_body,
        grid_spec=grid_spec,
        out_shape=jax.ShapeDtypeStruct((_B, _H, _W, _C), jnp.float32),
    )(perm, x, x, a, b)
    return y
